# baseline (speedup 1.0000x reference)
import functools

import jax
import jax.numpy as jnp
from jax import lax
from jax.experimental import pallas as pl
from jax.experimental.pallas import tpu as pltpu
from jax.experimental.pallas import tpu_sc as plsc

VOCAB = 100000
HIDDEN = 128
BATCH = 4096
CTX = 20

NUM_CORES = 2
NUM_SUBCORES = 16
NUM_WORKERS = NUM_CORES * NUM_SUBCORES
BPW = BATCH // NUM_WORKERS
LANES = 16
HCHUNKS = HIDDEN // LANES


def _sc_gather_sum(idx_t, emb_table):
    mesh = plsc.VectorSubcoreMesh(core_axis_name="c", subcore_axis_name="s")

    @functools.partial(
        pl.kernel,
        out_type=jax.ShapeDtypeStruct((BATCH, HIDDEN), jnp.float32),
        mesh=mesh,
        scratch_types=[
            pltpu.VMEM((CTX, BPW), jnp.int32),
            pltpu.VMEM((2, BPW, HIDDEN), jnp.float32),
            pltpu.VMEM((BPW, HIDDEN), jnp.float32),
            pltpu.SemaphoreType.DMA,
            pltpu.SemaphoreType.DMA,
            pltpu.SemaphoreType.DMA,
        ],
    )
    def k(idx_hbm, table_hbm, out_hbm, idx_v, rows_v, acc_v, sem0, semA, semB):
        wid = lax.axis_index("s") * NUM_CORES + lax.axis_index("c")
        base = wid * BPW
        pltpu.sync_copy(idx_hbm.at[:, pl.ds(base, BPW)], idx_v)
        sems = (semA, semB)
        cp0 = pltpu.async_copy(table_hbm.at[idx_v.at[0]], acc_v, sem0)
        pending = pltpu.async_copy(
            table_hbm.at[idx_v.at[1]], rows_v.at[0], sems[0])
        cp0.wait()
        for c in range(1, CTX):
            buf = (c - 1) % 2
            if c + 1 < CTX:
                nxt = pltpu.async_copy(
                    table_hbm.at[idx_v.at[c + 1]], rows_v.at[c % 2],
                    sems[c % 2])
            pending.wait()
            if c + 1 < CTX:
                pending = nxt

            @plsc.parallel_loop(0, BPW, 1, unroll=2)
            def row_step(i):
                for h in range(HCHUNKS):
                    sl = pl.ds(h * LANES, LANES)
                    plsc.addupdate(acc_v.at[i, sl], rows_v[buf, i, sl])

        pltpu.sync_copy(acc_v, out_hbm.at[pl.ds(base, BPW)])

    return k(idx_t, emb_table)


BM = 1024
BN = 2048
GI = BATCH // BM
GJ = (VOCAB + BN - 1) // BN
NBUF = 4
LAST = GI * GJ - 1
TAIL_A = ((VOCAB % BN) // 128) * 128
NTAIL = VOCAB % 128
MAIN_N = VOCAB - NTAIL


def _ring_desc(o_hbm, o_buf, sems, s, i, j, width):
    return pltpu.make_async_copy(
        o_buf.at[s, :, pl.ds(0, width)],
        o_hbm.at[pl.ds(i * BM, BM), pl.ds(j * BN, width)],
        sems.at[s],
    )


def _mm_body(s_ref, w_ref, b_ref, o_hbm, o_buf, sems):
    i = pl.program_id(0)
    j = pl.program_id(1)
    step = i * GJ + j
    slot = lax.rem(step, NBUF)

    acc = lax.dot_general(
        s_ref[...],
        w_ref[...],
        (((1,), (1,)), ((), ())),
        preferred_element_type=jnp.float32,
    ) + b_ref[0].astype(jnp.float32)

    @pl.when(step >= NBUF)
    def _():
        @pl.when(j == (GJ - 1 + NBUF) % GJ)
        def _():
            _ring_desc(o_hbm, o_buf, sems, slot, i, j, TAIL_A).wait()

        @pl.when(j != (GJ - 1 + NBUF) % GJ)
        def _():
            _ring_desc(o_hbm, o_buf, sems, slot, i, j, BN).wait()

    o_buf[slot] = acc

    @pl.when(j == GJ - 1)
    def _():
        _ring_desc(o_hbm, o_buf, sems, slot, i, j, TAIL_A).start()

    @pl.when(j != GJ - 1)
    def _():
        _ring_desc(o_hbm, o_buf, sems, slot, i, j, BN).start()

    @pl.when(step == LAST)
    def _():
        for st in range(LAST - NBUF + 1, LAST + 1):
            width = TAIL_A if st % GJ == GJ - 1 else BN
            _ring_desc(o_hbm, o_buf, sems, st % NBUF, i, j, width).wait()


def _tc_matmul(summed, out_w, out_bp):
    return pl.pallas_call(
        _mm_body,
        grid=(GI, GJ),
        in_specs=[
            pl.BlockSpec((BM, HIDDEN), lambda i, j: (i, 0)),
            pl.BlockSpec((BN, HIDDEN), lambda i, j: (j, 0)),
            pl.BlockSpec((1, 1, BN), lambda i, j: (j, 0, 0)),
        ],
        out_specs=pl.BlockSpec(memory_space=pl.ANY),
        out_shape=jax.ShapeDtypeStruct((BATCH, VOCAB), jnp.float32),
        scratch_shapes=[
            pltpu.VMEM((NBUF, BM, BN), jnp.float32),
            pltpu.SemaphoreType.DMA((NBUF,)),
        ],
        compiler_params=pltpu.CompilerParams(
            dimension_semantics=("arbitrary", "arbitrary"),
        ),
    )(summed, out_w, out_bp)


def _mm_tail_body(s_ref, w_ref, b_ref, o_ref):
    o_ref[...] = lax.dot_general(
        s_ref[...],
        w_ref[...],
        (((1,), (1,)), ((), ())),
        preferred_element_type=jnp.float32,
    ) + b_ref[...].astype(jnp.float32)


def _tc_matmul_tail(summed, w_tail, b_tail):
    return pl.pallas_call(
        _mm_tail_body,
        grid=(GI,),
        in_specs=[
            pl.BlockSpec((BM, HIDDEN), lambda i: (i, 0)),
            pl.BlockSpec((NTAIL, HIDDEN), lambda i: (0, 0)),
            pl.BlockSpec((1, NTAIL), lambda i: (0, 0)),
        ],
        out_specs=pl.BlockSpec((BM, NTAIL), lambda i: (i, 0)),
        out_shape=jax.ShapeDtypeStruct((BATCH, NTAIL), jnp.float32),
    )(summed, w_tail, b_tail)


def kernel(inputs, emb_table, out_w, out_b):
    idx_t = inputs.T.reshape(CTX, BATCH)
    summed = _sc_gather_sum(idx_t, emb_table)
    summed_bf = summed.astype(jnp.bfloat16)
    w_bf = out_w.astype(jnp.bfloat16)
    out_bp = jnp.pad(out_b, (0, GJ * BN - VOCAB)).reshape(GJ, 1, BN)
    logits = _tc_matmul(summed_bf, w_bf, out_bp)
    tail = _tc_matmul_tail(
        summed_bf,
        w_bf[MAIN_N:],
        out_b[MAIN_N:].reshape(1, NTAIL),
    )
    return lax.dynamic_update_slice(logits, tail, (0, MAIN_N))
